# baseline (speedup 1.0000x reference)
import functools

import jax
import jax.numpy as jnp
from jax import lax
from jax.experimental import pallas as pl
from jax.experimental.pallas import tpu as pltpu
from jax.experimental.pallas import tpu_sc as plsc

N_FIELDS = 26
VOCAB = 100000
EMB_DIM = 16
N_CONT = 13
BATCH = 16384
EMB_FEATS = N_FIELDS * EMB_DIM

_NW = 32
_B_PER_W = BATCH // _NW
_CHUNK_B = 128
_CHUNK_R = _CHUNK_B * N_FIELDS
_N_CHUNKS = _B_PER_W // _CHUNK_B


@functools.cache
def _make_sc_gather():
    info = plsc.get_sparse_core_info()
    num_cores = info.num_cores

    @functools.partial(
        pl.kernel,
        out_type=jax.ShapeDtypeStruct((BATCH * N_FIELDS, EMB_DIM), jnp.float32),
        mesh=plsc.VectorSubcoreMesh(core_axis_name="c", subcore_axis_name="s"),
        scratch_types=[
            pltpu.VMEM((_CHUNK_R,), jnp.int32),
            pltpu.VMEM((_CHUNK_R, EMB_DIM), jnp.float32),
            pltpu.SemaphoreType.DMA,
        ],
        compiler_params=pltpu.CompilerParams(use_tc_tiling_on_sc=False),
    )
    def _sc_gather(table_hbm, idx_hbm, out_hbm, idx_v, rows_v, sem):
        wid = lax.axis_index("s") * num_cores + lax.axis_index("c")
        base = wid * (_B_PER_W * N_FIELDS)
        for g in range(_N_CHUNKS):
            off = base + g * _CHUNK_R
            pltpu.sync_copy(idx_hbm.at[pl.ds(off, _CHUNK_R)], idx_v)
            pltpu.async_copy(table_hbm.at[idx_v], rows_v, sem).wait()
            pltpu.sync_copy(rows_v, out_hbm.at[pl.ds(off, _CHUNK_R)])

    return _sc_gather


def _mlp_body(xe_ref, xc_ref, w0e_ref, w0c_ref, b0_ref, w1_ref, b1_ref,
              w2_ref, b2_ref, wh_ref, bh_ref, out_ref):
    h = jnp.dot(xe_ref[...], w0e_ref[...], preferred_element_type=jnp.float32)
    h += jnp.dot(xc_ref[...], w0c_ref[...], preferred_element_type=jnp.float32)
    h = jnp.maximum(h + b0_ref[...], 0.0)
    h = jnp.maximum(
        jnp.dot(h, w1_ref[...], preferred_element_type=jnp.float32) + b1_ref[...], 0.0)
    h = jnp.maximum(
        jnp.dot(h, w2_ref[...], preferred_element_type=jnp.float32) + b2_ref[...], 0.0)
    out_ref[...] = jnp.dot(h, wh_ref[...], preferred_element_type=jnp.float32) + bh_ref[...]


_BT = 1024


def _mlp(xe, xc, w0e, w0c, b0, w1, b1, w2, b2, wh, bh):
    n_blocks = BATCH // _BT
    full = lambda shape: pl.BlockSpec(shape, lambda i: (0, 0))
    return pl.pallas_call(
        _mlp_body,
        grid=(n_blocks,),
        in_specs=[
            pl.BlockSpec((_BT, EMB_FEATS), lambda i: (i, 0)),
            pl.BlockSpec((_BT, N_CONT), lambda i: (i, 0)),
            full((EMB_FEATS, 512)),
            full((N_CONT, 512)),
            full((1, 512)),
            full((512, 256)),
            full((1, 256)),
            full((256, 128)),
            full((1, 128)),
            full((128, 1)),
            full((1, 1)),
        ],
        out_specs=pl.BlockSpec((_BT, 1), lambda i: (i, 0)),
        out_shape=jax.ShapeDtypeStruct((BATCH, 1), jnp.float32),
    )(xe, xc, w0e, w0c, b0, w1, b1, w2, b2, wh, bh)


def kernel(x_cont, x_cat, emb_tables, W0, b0, W1, b1, W2, b2, Wh, bh):
    table = emb_tables.reshape(N_FIELDS * VOCAB, EMB_DIM)
    offs = jnp.arange(N_FIELDS, dtype=jnp.int32) * VOCAB
    idx = (x_cat.astype(jnp.int32) + offs[None, :]).reshape(-1)
    rows = _make_sc_gather()(table, idx)
    return jnp.sum(rows.reshape(BATCH, EMB_FEATS), axis=1, keepdims=True)
    xe = rows.reshape(BATCH, EMB_FEATS)
    return _mlp(xe, x_cont, W0[N_CONT:], W0[:N_CONT],
                b0.reshape(1, -1), W1, b1.reshape(1, -1),
                W2, b2.reshape(1, -1), Wh, bh.reshape(1, 1))


# speedup vs baseline: 2.7402x; 2.7402x over previous
import functools

import jax
import jax.numpy as jnp
from jax import lax
from jax.experimental import pallas as pl
from jax.experimental.pallas import tpu as pltpu
from jax.experimental.pallas import tpu_sc as plsc

N_FIELDS = 26
VOCAB = 100000
EMB_DIM = 16
N_CONT = 13
BATCH = 16384
EMB_FEATS = N_FIELDS * EMB_DIM

_NW = 32

_VS = 1536
_NFULL = 64
_TAIL1 = 1536
_TAIL2 = 128
_REM0 = 99968
_NREM = VOCAB - _REM0

_B_PER_W = BATCH // _NW
_CHUNK_B = 128
_CHUNK_R = _CHUNK_B * N_FIELDS
_N_CHUNKS = _B_PER_W // _CHUNK_B


@functools.cache
def _make_sc_transpose():
    @functools.partial(
        pl.kernel,
        out_type=jax.ShapeDtypeStruct((N_FIELDS * VOCAB * EMB_DIM,), jnp.float32),
        mesh=plsc.VectorSubcoreMesh(core_axis_name="c", subcore_axis_name="s"),
        scratch_types=[
            pltpu.VMEM((EMB_DIM, _VS), jnp.float32),
            pltpu.VMEM((EMB_DIM, _VS), jnp.float32),
            pltpu.VMEM((_VS * EMB_DIM,), jnp.float32),
            pltpu.VMEM((_VS * EMB_DIM,), jnp.float32),
            pltpu.SemaphoreType.DMA,
            pltpu.SemaphoreType.DMA,
            pltpu.SemaphoreType.DMA,
            pltpu.SemaphoreType.DMA,
        ],
        compiler_params=pltpu.CompilerParams(use_tc_tiling_on_sc=True,
                                             needs_layout_passes=False),
    )
    def _t(t2_hbm, tail_hbm, out_hbm, slab0, slab1, outf0, outf1,
           si0, si1, so0, so1):
        wid = lax.axis_index("s") * 2 + lax.axis_index("c")
        lane = lax.iota(jnp.int32, 16)
        lane16 = lane * 16
        v0 = wid * _VS
        v1 = (wid + 32) * _VS
        FB = VOCAB * EMB_DIM

        def transpose_groups(slab, outf, n_groups):
            def body(g, _):
                vidx = lane + g * 16
                for e in range(EMB_DIM):
                    rvec = plsc.load_gather(
                        slab, [jnp.full((16,), e, jnp.int32), vidx])
                    plsc.store_scatter(outf, [lane16 + (g * 256 + e)], rvec)
                return 0
            lax.fori_loop(0, n_groups, body, 0)

        def start_in(kk, vv, slab, sem):
            pltpu.async_copy(
                t2_hbm.at[pl.ds(kk * 16, 16), pl.ds(vv, _VS)], slab, sem)

        def wait_in(slab, sem):
            pltpu.make_async_copy(
                t2_hbm.at[pl.ds(0, 16), pl.ds(0, _VS)], slab, sem).wait()

        def wait_out(outf, sem):
            pltpu.make_async_copy(
                outf, out_hbm.at[pl.ds(0, _VS * EMB_DIM)], sem).wait()

        start_in(0, v0, slab0, si0)

        def body(jj, _):
            start_in(jj, v1, slab1, si1)
            wait_in(slab0, si0)
            @pl.when(jj > 0)
            def _():
                wait_out(outf0, so0)
            transpose_groups(slab0, outf0, _VS // 16)
            pltpu.async_copy(
                outf0, out_hbm.at[pl.ds(jj * FB + v0 * EMB_DIM,
                                        _VS * EMB_DIM)], so0)
            @pl.when(jj < N_FIELDS - 1)
            def _():
                start_in(jj + 1, v0, slab0, si0)
            wait_in(slab1, si1)
            @pl.when(jj > 0)
            def _():
                wait_out(outf1, so1)
            transpose_groups(slab1, outf1, _VS // 16)
            pltpu.async_copy(
                outf1, out_hbm.at[pl.ds(jj * FB + v1 * EMB_DIM,
                                        _VS * EMB_DIM)], so1)
            return 0

        lax.fori_loop(0, N_FIELDS, body, 0)
        wait_out(outf0, so0)
        wait_out(outf1, so1)

        @pl.when(wid < N_FIELDS)
        def _():
            base = wid * FB
            r0 = wid * 16
            pltpu.sync_copy(t2_hbm.at[pl.ds(r0, 16), pl.ds(98304, _TAIL1)],
                            slab0)
            transpose_groups(slab0, outf0, _TAIL1 // 16)
            pltpu.sync_copy(outf0,
                            out_hbm.at[pl.ds(base + 98304 * EMB_DIM,
                                             _TAIL1 * EMB_DIM)])
            pltpu.sync_copy(t2_hbm.at[pl.ds(r0, 16), pl.ds(99840, _TAIL2)],
                            slab1.at[:, pl.ds(0, _TAIL2)])
            transpose_groups(slab1, outf1, _TAIL2 // 16)
            pltpu.sync_copy(outf1.at[pl.ds(0, _TAIL2 * EMB_DIM)],
                            out_hbm.at[pl.ds(base + 99840 * EMB_DIM,
                                             _TAIL2 * EMB_DIM)])
            n = _NREM * EMB_DIM
            pltpu.sync_copy(tail_hbm.at[pl.ds(wid * n, n)],
                            outf0.at[pl.ds(0, n)])
            pltpu.sync_copy(outf0.at[pl.ds(0, n)],
                            out_hbm.at[pl.ds(base + _REM0 * EMB_DIM, n)])

    return _t


@functools.cache
def _make_sc_gather():
    @functools.partial(
        pl.kernel,
        out_type=jax.ShapeDtypeStruct((BATCH * N_FIELDS, EMB_DIM), jnp.float32),
        mesh=plsc.VectorSubcoreMesh(core_axis_name="c", subcore_axis_name="s"),
        scratch_types=[
            pltpu.VMEM((_CHUNK_R,), jnp.int32),
            pltpu.VMEM((_CHUNK_R, EMB_DIM), jnp.float32),
            pltpu.SemaphoreType.DMA,
        ],
        compiler_params=pltpu.CompilerParams(use_tc_tiling_on_sc=False),
    )
    def _sc_gather(table_hbm, idx_hbm, out_hbm, idx_v, rows_v, sem):
        wid = lax.axis_index("s") * 2 + lax.axis_index("c")
        base = wid * (_B_PER_W * N_FIELDS)
        for g in range(_N_CHUNKS):
            off = base + g * _CHUNK_R
            pltpu.sync_copy(idx_hbm.at[pl.ds(off, _CHUNK_R)], idx_v)
            pltpu.async_copy(table_hbm.at[idx_v], rows_v, sem).wait()
            pltpu.sync_copy(rows_v, out_hbm.at[pl.ds(off, _CHUNK_R)])

    return _sc_gather


def _mlp_body(xe_ref, xc_ref, w0e_ref, w0c_ref, b0_ref, w1_ref, b1_ref,
              w2_ref, b2_ref, wh_ref, bh_ref, out_ref):
    h = jnp.dot(xe_ref[...], w0e_ref[...], preferred_element_type=jnp.float32)
    h += jnp.dot(xc_ref[...], w0c_ref[...], preferred_element_type=jnp.float32)
    h = jnp.maximum(h + b0_ref[...], 0.0)
    h = jnp.maximum(
        jnp.dot(h, w1_ref[...], preferred_element_type=jnp.float32) + b1_ref[...], 0.0)
    h = jnp.maximum(
        jnp.dot(h, w2_ref[...], preferred_element_type=jnp.float32) + b2_ref[...], 0.0)
    out_ref[...] = jnp.dot(h, wh_ref[...], preferred_element_type=jnp.float32) + bh_ref[...]


_BT = 1024


def _mlp(xe, xc, w0e, w0c, b0, w1, b1, w2, b2, wh, bh):
    n_blocks = BATCH // _BT
    full = lambda shape: pl.BlockSpec(shape, lambda i: (0, 0))
    return pl.pallas_call(
        _mlp_body,
        grid=(n_blocks,),
        in_specs=[
            pl.BlockSpec((_BT, EMB_FEATS), lambda i: (i, 0)),
            pl.BlockSpec((_BT, N_CONT), lambda i: (i, 0)),
            full((EMB_FEATS, 512)),
            full((N_CONT, 512)),
            full((1, 512)),
            full((512, 256)),
            full((1, 256)),
            full((256, 128)),
            full((1, 128)),
            full((128, 1)),
            full((1, 1)),
        ],
        out_specs=pl.BlockSpec((_BT, 1), lambda i: (i, 0)),
        out_shape=jax.ShapeDtypeStruct((BATCH, 1), jnp.float32),
    )(xe, xc, w0e, w0c, b0, w1, b1, w2, b2, wh, bh)


def kernel(x_cont, x_cat, emb_tables, W0, b0, W1, b1, W2, b2, Wh, bh):
    t2 = jnp.transpose(emb_tables, (0, 2, 1)).reshape(N_FIELDS * EMB_DIM, VOCAB)
    tail = emb_tables[:, _REM0:, :].reshape(-1)
    flat = _make_sc_transpose()(t2, tail)
    table = flat.reshape(N_FIELDS * VOCAB, EMB_DIM)
    offs = jnp.arange(N_FIELDS, dtype=jnp.int32) * VOCAB
    idx = (x_cat.astype(jnp.int32) + offs[None, :]).reshape(-1)
    rows = _make_sc_gather()(table, idx)
    xe = rows.reshape(BATCH, EMB_FEATS)
    return _mlp(xe, x_cont, W0[N_CONT:], W0[:N_CONT],
                b0.reshape(1, -1), W1, b1.reshape(1, -1),
                W2, b2.reshape(1, -1), Wh, bh.reshape(1, 1))


# speedup vs baseline: 2.7514x; 1.0041x over previous
import functools

import jax
import jax.numpy as jnp
from jax import lax
from jax.experimental import pallas as pl
from jax.experimental.pallas import tpu as pltpu
from jax.experimental.pallas import tpu_sc as plsc

N_FIELDS = 26
VOCAB = 100000
EMB_DIM = 16
N_CONT = 13
BATCH = 16384
EMB_FEATS = N_FIELDS * EMB_DIM

_NW = 32

_VS = 1536
_NFULL = 64
_TAIL1 = 1536
_TAIL2 = 128
_REM0 = 99968
_NREM = VOCAB - _REM0

_B_PER_W = BATCH // _NW
_CHUNK_B = 128
_CHUNK_R = _CHUNK_B * N_FIELDS
_N_CHUNKS = _B_PER_W // _CHUNK_B


@functools.cache
def _make_sc_transpose():
    @functools.partial(
        pl.kernel,
        out_type=jax.ShapeDtypeStruct((N_FIELDS * VOCAB * EMB_DIM,), jnp.float32),
        mesh=plsc.VectorSubcoreMesh(core_axis_name="c", subcore_axis_name="s"),
        scratch_types=[
            pltpu.VMEM((EMB_DIM, _VS), jnp.float32),
            pltpu.VMEM((EMB_DIM, _VS), jnp.float32),
            pltpu.VMEM((_VS * EMB_DIM,), jnp.float32),
            pltpu.VMEM((_VS * EMB_DIM,), jnp.float32),
            pltpu.SemaphoreType.DMA,
            pltpu.SemaphoreType.DMA,
            pltpu.SemaphoreType.DMA,
            pltpu.SemaphoreType.DMA,
        ],
        compiler_params=pltpu.CompilerParams(use_tc_tiling_on_sc=True,
                                             needs_layout_passes=False),
    )
    def _t(t2_hbm, tail_hbm, out_hbm, slab0, slab1, outf0, outf1,
           si0, si1, so0, so1):
        wid = lax.axis_index("s") * 2 + lax.axis_index("c")
        lane = lax.iota(jnp.int32, 16)
        lane16 = lane * 16
        v0 = wid * _VS
        v1 = (wid + 32) * _VS
        FB = VOCAB * EMB_DIM

        def transpose_groups(slab, outf, n_groups):
            def body(g, _):
                vidx = lane + g * 16
                for e in range(EMB_DIM):
                    rvec = plsc.load_gather(
                        slab, [jnp.full((16,), e, jnp.int32), vidx])
                    plsc.store_scatter(outf, [lane16 + (g * 256 + e)], rvec)
                return 0
            lax.fori_loop(0, n_groups, body, 0, unroll=4)

        def start_in(kk, vv, slab, sem):
            pltpu.async_copy(
                t2_hbm.at[pl.ds(kk * 16, 16), pl.ds(vv, _VS)], slab, sem)

        def wait_in(slab, sem):
            pltpu.make_async_copy(
                t2_hbm.at[pl.ds(0, 16), pl.ds(0, _VS)], slab, sem).wait()

        def wait_out(outf, sem):
            pltpu.make_async_copy(
                outf, out_hbm.at[pl.ds(0, _VS * EMB_DIM)], sem).wait()

        start_in(0, v0, slab0, si0)

        def body(jj, _):
            start_in(jj, v1, slab1, si1)
            wait_in(slab0, si0)
            @pl.when(jj > 0)
            def _():
                wait_out(outf0, so0)
            transpose_groups(slab0, outf0, _VS // 16)
            pltpu.async_copy(
                outf0, out_hbm.at[pl.ds(jj * FB + v0 * EMB_DIM,
                                        _VS * EMB_DIM)], so0)
            @pl.when(jj < N_FIELDS - 1)
            def _():
                start_in(jj + 1, v0, slab0, si0)
            wait_in(slab1, si1)
            @pl.when(jj > 0)
            def _():
                wait_out(outf1, so1)
            transpose_groups(slab1, outf1, _VS // 16)
            pltpu.async_copy(
                outf1, out_hbm.at[pl.ds(jj * FB + v1 * EMB_DIM,
                                        _VS * EMB_DIM)], so1)
            return 0

        lax.fori_loop(0, N_FIELDS, body, 0)
        wait_out(outf0, so0)
        wait_out(outf1, so1)

        @pl.when(wid < N_FIELDS)
        def _():
            base = wid * FB
            r0 = wid * 16
            pltpu.sync_copy(t2_hbm.at[pl.ds(r0, 16), pl.ds(98304, _TAIL1)],
                            slab0)
            transpose_groups(slab0, outf0, _TAIL1 // 16)
            pltpu.sync_copy(outf0,
                            out_hbm.at[pl.ds(base + 98304 * EMB_DIM,
                                             _TAIL1 * EMB_DIM)])
            pltpu.sync_copy(t2_hbm.at[pl.ds(r0, 16), pl.ds(99840, _TAIL2)],
                            slab1.at[:, pl.ds(0, _TAIL2)])
            transpose_groups(slab1, outf1, _TAIL2 // 16)
            pltpu.sync_copy(outf1.at[pl.ds(0, _TAIL2 * EMB_DIM)],
                            out_hbm.at[pl.ds(base + 99840 * EMB_DIM,
                                             _TAIL2 * EMB_DIM)])
            n = _NREM * EMB_DIM
            pltpu.sync_copy(tail_hbm.at[pl.ds(wid * n, n)],
                            outf0.at[pl.ds(0, n)])
            pltpu.sync_copy(outf0.at[pl.ds(0, n)],
                            out_hbm.at[pl.ds(base + _REM0 * EMB_DIM, n)])

    return _t


@functools.cache
def _make_sc_gather():
    @functools.partial(
        pl.kernel,
        out_type=jax.ShapeDtypeStruct((BATCH * N_FIELDS, EMB_DIM), jnp.float32),
        mesh=plsc.VectorSubcoreMesh(core_axis_name="c", subcore_axis_name="s"),
        scratch_types=[
            pltpu.VMEM((_CHUNK_R,), jnp.int32),
            pltpu.VMEM((_CHUNK_R, EMB_DIM), jnp.float32),
            pltpu.SemaphoreType.DMA,
        ],
        compiler_params=pltpu.CompilerParams(use_tc_tiling_on_sc=False),
    )
    def _sc_gather(table_hbm, idx_hbm, out_hbm, idx_v, rows_v, sem):
        wid = lax.axis_index("s") * 2 + lax.axis_index("c")
        base = wid * (_B_PER_W * N_FIELDS)
        for g in range(_N_CHUNKS):
            off = base + g * _CHUNK_R
            pltpu.sync_copy(idx_hbm.at[pl.ds(off, _CHUNK_R)], idx_v)
            pltpu.async_copy(table_hbm.at[idx_v], rows_v, sem).wait()
            pltpu.sync_copy(rows_v, out_hbm.at[pl.ds(off, _CHUNK_R)])

    return _sc_gather


def _mlp_body(xe_ref, xc_ref, w0e_ref, w0c_ref, b0_ref, w1_ref, b1_ref,
              w2_ref, b2_ref, wh_ref, bh_ref, out_ref):
    h = jnp.dot(xe_ref[...], w0e_ref[...], preferred_element_type=jnp.float32)
    h += jnp.dot(xc_ref[...], w0c_ref[...], preferred_element_type=jnp.float32)
    h = jnp.maximum(h + b0_ref[...], 0.0)
    h = jnp.maximum(
        jnp.dot(h, w1_ref[...], preferred_element_type=jnp.float32) + b1_ref[...], 0.0)
    h = jnp.maximum(
        jnp.dot(h, w2_ref[...], preferred_element_type=jnp.float32) + b2_ref[...], 0.0)
    out_ref[...] = jnp.dot(h, wh_ref[...], preferred_element_type=jnp.float32) + bh_ref[...]


_BT = 1024


def _mlp(xe, xc, w0e, w0c, b0, w1, b1, w2, b2, wh, bh):
    n_blocks = BATCH // _BT
    full = lambda shape: pl.BlockSpec(shape, lambda i: (0, 0))
    return pl.pallas_call(
        _mlp_body,
        grid=(n_blocks,),
        in_specs=[
            pl.BlockSpec((_BT, EMB_FEATS), lambda i: (i, 0)),
            pl.BlockSpec((_BT, N_CONT), lambda i: (i, 0)),
            full((EMB_FEATS, 512)),
            full((N_CONT, 512)),
            full((1, 512)),
            full((512, 256)),
            full((1, 256)),
            full((256, 128)),
            full((1, 128)),
            full((128, 1)),
            full((1, 1)),
        ],
        out_specs=pl.BlockSpec((_BT, 1), lambda i: (i, 0)),
        out_shape=jax.ShapeDtypeStruct((BATCH, 1), jnp.float32),
    )(xe, xc, w0e, w0c, b0, w1, b1, w2, b2, wh, bh)


def kernel(x_cont, x_cat, emb_tables, W0, b0, W1, b1, W2, b2, Wh, bh):
    t2 = jnp.transpose(emb_tables, (0, 2, 1)).reshape(N_FIELDS * EMB_DIM, VOCAB)
    tail = emb_tables[:, _REM0:, :].reshape(-1)
    flat = _make_sc_transpose()(t2, tail)
    table = flat.reshape(N_FIELDS * VOCAB, EMB_DIM)
    offs = jnp.arange(N_FIELDS, dtype=jnp.int32) * VOCAB
    idx = (x_cat.astype(jnp.int32) + offs[None, :]).reshape(-1)
    rows = _make_sc_gather()(table, idx)
    xe = rows.reshape(BATCH, EMB_FEATS)
    return _mlp(xe, x_cont, W0[N_CONT:], W0[:N_CONT],
                b0.reshape(1, -1), W1, b1.reshape(1, -1),
                W2, b2.reshape(1, -1), Wh, bh.reshape(1, 1))


# speedup vs baseline: 3.0409x; 1.1052x over previous
import functools

import jax
import jax.numpy as jnp
from jax import lax
from jax.experimental import pallas as pl
from jax.experimental.pallas import tpu as pltpu
from jax.experimental.pallas import tpu_sc as plsc

N_FIELDS = 26
VOCAB = 100000
EMB_DIM = 16
N_CONT = 13
BATCH = 16384
EMB_FEATS = N_FIELDS * EMB_DIM

_NW = 32

_VS = 1536
_NFULL = 64
_TAIL1 = 1536
_TAIL2 = 128
_REM0 = 99968
_NREM = VOCAB - _REM0

_B_PER_W = BATCH // _NW
_CHUNK_B = 128
_CHUNK_R = _CHUNK_B * N_FIELDS
_N_CHUNKS = _B_PER_W // _CHUNK_B


@functools.cache
def _make_sc_transpose():
    @functools.partial(
        pl.kernel,
        out_type=jax.ShapeDtypeStruct((N_FIELDS * VOCAB * EMB_DIM,), jnp.float32),
        mesh=plsc.VectorSubcoreMesh(core_axis_name="c", subcore_axis_name="s"),
        scratch_types=[
            pltpu.VMEM((EMB_DIM, _VS), jnp.float32),
            pltpu.VMEM((EMB_DIM, _VS), jnp.float32),
            pltpu.VMEM((_VS * EMB_DIM,), jnp.float32),
            pltpu.VMEM((_VS * EMB_DIM,), jnp.float32),
            pltpu.SemaphoreType.DMA,
            pltpu.SemaphoreType.DMA,
            pltpu.SemaphoreType.DMA,
            pltpu.SemaphoreType.DMA,
        ],
        compiler_params=pltpu.CompilerParams(use_tc_tiling_on_sc=True,
                                             needs_layout_passes=False),
    )
    def _t(t2_hbm, tail_hbm, out_hbm, slab0, slab1, outf0, outf1,
           si0, si1, so0, so1):
        wid = lax.axis_index("s") * 2 + lax.axis_index("c")
        lane = lax.iota(jnp.int32, 16)
        lane16 = lane * 16
        v0 = wid * _VS
        v1 = (wid + 32) * _VS
        FB = VOCAB * EMB_DIM

        def transpose_groups(slab, outf, n_groups):
            def body(g, _):
                for e in range(EMB_DIM):
                    rvec = slab[e, pl.ds(g * 16, 16)]
                    plsc.store_scatter(outf, [lane16 + (g * 256 + e)], rvec)
                return 0
            lax.fori_loop(0, n_groups, body, 0, unroll=4)

        def start_in(kk, vv, slab, sem):
            pltpu.async_copy(
                t2_hbm.at[pl.ds(kk * 16, 16), pl.ds(vv, _VS)], slab, sem)

        def wait_in(slab, sem):
            pltpu.make_async_copy(
                t2_hbm.at[pl.ds(0, 16), pl.ds(0, _VS)], slab, sem).wait()

        def wait_out(outf, sem):
            pltpu.make_async_copy(
                outf, out_hbm.at[pl.ds(0, _VS * EMB_DIM)], sem).wait()

        start_in(0, v0, slab0, si0)

        def body(jj, _):
            start_in(jj, v1, slab1, si1)
            wait_in(slab0, si0)
            @pl.when(jj > 0)
            def _():
                wait_out(outf0, so0)
            transpose_groups(slab0, outf0, _VS // 16)
            pltpu.async_copy(
                outf0, out_hbm.at[pl.ds(jj * FB + v0 * EMB_DIM,
                                        _VS * EMB_DIM)], so0)
            @pl.when(jj < N_FIELDS - 1)
            def _():
                start_in(jj + 1, v0, slab0, si0)
            wait_in(slab1, si1)
            @pl.when(jj > 0)
            def _():
                wait_out(outf1, so1)
            transpose_groups(slab1, outf1, _VS // 16)
            pltpu.async_copy(
                outf1, out_hbm.at[pl.ds(jj * FB + v1 * EMB_DIM,
                                        _VS * EMB_DIM)], so1)
            return 0

        lax.fori_loop(0, N_FIELDS, body, 0)
        wait_out(outf0, so0)
        wait_out(outf1, so1)

        @pl.when(wid < N_FIELDS)
        def _():
            base = wid * FB
            r0 = wid * 16
            pltpu.sync_copy(t2_hbm.at[pl.ds(r0, 16), pl.ds(98304, _TAIL1)],
                            slab0)
            transpose_groups(slab0, outf0, _TAIL1 // 16)
            pltpu.sync_copy(outf0,
                            out_hbm.at[pl.ds(base + 98304 * EMB_DIM,
                                             _TAIL1 * EMB_DIM)])
            pltpu.sync_copy(t2_hbm.at[pl.ds(r0, 16), pl.ds(99840, _TAIL2)],
                            slab1.at[:, pl.ds(0, _TAIL2)])
            transpose_groups(slab1, outf1, _TAIL2 // 16)
            pltpu.sync_copy(outf1.at[pl.ds(0, _TAIL2 * EMB_DIM)],
                            out_hbm.at[pl.ds(base + 99840 * EMB_DIM,
                                             _TAIL2 * EMB_DIM)])
            n = _NREM * EMB_DIM
            pltpu.sync_copy(tail_hbm.at[pl.ds(wid * n, n)],
                            outf0.at[pl.ds(0, n)])
            pltpu.sync_copy(outf0.at[pl.ds(0, n)],
                            out_hbm.at[pl.ds(base + _REM0 * EMB_DIM, n)])

    return _t


@functools.cache
def _make_sc_gather():
    @functools.partial(
        pl.kernel,
        out_type=jax.ShapeDtypeStruct((BATCH * N_FIELDS, EMB_DIM), jnp.float32),
        mesh=plsc.VectorSubcoreMesh(core_axis_name="c", subcore_axis_name="s"),
        scratch_types=[
            pltpu.VMEM((_CHUNK_R,), jnp.int32),
            pltpu.VMEM((_CHUNK_R, EMB_DIM), jnp.float32),
            pltpu.SemaphoreType.DMA,
        ],
        compiler_params=pltpu.CompilerParams(use_tc_tiling_on_sc=False),
    )
    def _sc_gather(table_hbm, idx_hbm, out_hbm, idx_v, rows_v, sem):
        wid = lax.axis_index("s") * 2 + lax.axis_index("c")
        base = wid * (_B_PER_W * N_FIELDS)
        for g in range(_N_CHUNKS):
            off = base + g * _CHUNK_R
            pltpu.sync_copy(idx_hbm.at[pl.ds(off, _CHUNK_R)], idx_v)
            pltpu.async_copy(table_hbm.at[idx_v], rows_v, sem).wait()
            pltpu.sync_copy(rows_v, out_hbm.at[pl.ds(off, _CHUNK_R)])

    return _sc_gather


def _mlp_body(xe_ref, xc_ref, w0e_ref, w0c_ref, b0_ref, w1_ref, b1_ref,
              w2_ref, b2_ref, wh_ref, bh_ref, out_ref):
    h = jnp.dot(xe_ref[...], w0e_ref[...], preferred_element_type=jnp.float32)
    h += jnp.dot(xc_ref[...], w0c_ref[...], preferred_element_type=jnp.float32)
    h = jnp.maximum(h + b0_ref[...], 0.0)
    h = jnp.maximum(
        jnp.dot(h, w1_ref[...], preferred_element_type=jnp.float32) + b1_ref[...], 0.0)
    h = jnp.maximum(
        jnp.dot(h, w2_ref[...], preferred_element_type=jnp.float32) + b2_ref[...], 0.0)
    out_ref[...] = jnp.dot(h, wh_ref[...], preferred_element_type=jnp.float32) + bh_ref[...]


_BT = 1024


def _mlp(xe, xc, w0e, w0c, b0, w1, b1, w2, b2, wh, bh):
    n_blocks = BATCH // _BT
    full = lambda shape: pl.BlockSpec(shape, lambda i: (0, 0))
    return pl.pallas_call(
        _mlp_body,
        grid=(n_blocks,),
        in_specs=[
            pl.BlockSpec((_BT, EMB_FEATS), lambda i: (i, 0)),
            pl.BlockSpec((_BT, N_CONT), lambda i: (i, 0)),
            full((EMB_FEATS, 512)),
            full((N_CONT, 512)),
            full((1, 512)),
            full((512, 256)),
            full((1, 256)),
            full((256, 128)),
            full((1, 128)),
            full((128, 1)),
            full((1, 1)),
        ],
        out_specs=pl.BlockSpec((_BT, 1), lambda i: (i, 0)),
        out_shape=jax.ShapeDtypeStruct((BATCH, 1), jnp.float32),
    )(xe, xc, w0e, w0c, b0, w1, b1, w2, b2, wh, bh)


def kernel(x_cont, x_cat, emb_tables, W0, b0, W1, b1, W2, b2, Wh, bh):
    t2 = jnp.transpose(emb_tables, (0, 2, 1)).reshape(N_FIELDS * EMB_DIM, VOCAB)
    tail = emb_tables[:, _REM0:, :].reshape(-1)
    flat = _make_sc_transpose()(t2, tail)
    table = flat.reshape(N_FIELDS * VOCAB, EMB_DIM)
    offs = jnp.arange(N_FIELDS, dtype=jnp.int32) * VOCAB
    idx = (x_cat.astype(jnp.int32) + offs[None, :]).reshape(-1)
    rows = _make_sc_gather()(table, idx)
    xe = rows.reshape(BATCH, EMB_FEATS)
    return _mlp(xe, x_cont, W0[N_CONT:], W0[:N_CONT],
                b0.reshape(1, -1), W1, b1.reshape(1, -1),
                W2, b2.reshape(1, -1), Wh, bh.reshape(1, 1))


# speedup vs baseline: 3.0441x; 1.0011x over previous
import functools

import jax
import jax.numpy as jnp
from jax import lax
from jax.experimental import pallas as pl
from jax.experimental.pallas import tpu as pltpu
from jax.experimental.pallas import tpu_sc as plsc

N_FIELDS = 26
VOCAB = 100000
EMB_DIM = 16
N_CONT = 13
BATCH = 16384
EMB_FEATS = N_FIELDS * EMB_DIM

_NW = 32

_VS = 1536
_NFULL = 64
_TAIL1 = 1536
_TAIL2 = 128
_REM0 = 99968
_NREM = VOCAB - _REM0

_B_PER_W = BATCH // _NW
_CHUNK_B = 128
_CHUNK_R = _CHUNK_B * N_FIELDS
_N_CHUNKS = _B_PER_W // _CHUNK_B


@functools.cache
def _make_sc_transpose():
    @functools.partial(
        pl.kernel,
        out_type=jax.ShapeDtypeStruct((N_FIELDS * VOCAB * EMB_DIM,), jnp.float32),
        mesh=plsc.VectorSubcoreMesh(core_axis_name="c", subcore_axis_name="s"),
        scratch_types=[
            pltpu.VMEM((EMB_DIM, _VS), jnp.float32),
            pltpu.VMEM((EMB_DIM, _VS), jnp.float32),
            pltpu.VMEM((_VS * EMB_DIM,), jnp.float32),
            pltpu.VMEM((_VS * EMB_DIM,), jnp.float32),
            pltpu.SemaphoreType.DMA,
            pltpu.SemaphoreType.DMA,
            pltpu.SemaphoreType.DMA,
            pltpu.SemaphoreType.DMA,
        ],
        compiler_params=pltpu.CompilerParams(use_tc_tiling_on_sc=True,
                                             needs_layout_passes=False),
    )
    def _t(t2_hbm, tail_hbm, out_hbm, slab0, slab1, outf0, outf1,
           si0, si1, so0, so1):
        wid = lax.axis_index("s") * 2 + lax.axis_index("c")
        lane = lax.iota(jnp.int32, 16)
        lane16 = lane * 16
        v0 = wid * _VS
        v1 = (wid + 32) * _VS
        FB = VOCAB * EMB_DIM

        def transpose_groups(slab, outf, n_groups):
            def body(g, _):
                for e in range(EMB_DIM):
                    rvec = slab[e, pl.ds(g * 16, 16)]
                    plsc.store_scatter(outf, [lane16 + (g * 256 + e)], rvec)
                return 0
            lax.fori_loop(0, n_groups, body, 0, unroll=8)

        def start_in(kk, vv, slab, sem):
            pltpu.async_copy(
                t2_hbm.at[pl.ds(kk * 16, 16), pl.ds(vv, _VS)], slab, sem)

        def wait_in(slab, sem):
            pltpu.make_async_copy(
                t2_hbm.at[pl.ds(0, 16), pl.ds(0, _VS)], slab, sem).wait()

        def wait_out(outf, sem):
            pltpu.make_async_copy(
                outf, out_hbm.at[pl.ds(0, _VS * EMB_DIM)], sem).wait()

        start_in(0, v0, slab0, si0)

        def body(jj, _):
            start_in(jj, v1, slab1, si1)
            wait_in(slab0, si0)
            @pl.when(jj > 0)
            def _():
                wait_out(outf0, so0)
            transpose_groups(slab0, outf0, _VS // 16)
            pltpu.async_copy(
                outf0, out_hbm.at[pl.ds(jj * FB + v0 * EMB_DIM,
                                        _VS * EMB_DIM)], so0)
            @pl.when(jj < N_FIELDS - 1)
            def _():
                start_in(jj + 1, v0, slab0, si0)
            wait_in(slab1, si1)
            @pl.when(jj > 0)
            def _():
                wait_out(outf1, so1)
            transpose_groups(slab1, outf1, _VS // 16)
            pltpu.async_copy(
                outf1, out_hbm.at[pl.ds(jj * FB + v1 * EMB_DIM,
                                        _VS * EMB_DIM)], so1)
            return 0

        lax.fori_loop(0, N_FIELDS, body, 0)
        wait_out(outf0, so0)
        wait_out(outf1, so1)

        @pl.when(wid < N_FIELDS)
        def _():
            base = wid * FB
            r0 = wid * 16
            pltpu.sync_copy(t2_hbm.at[pl.ds(r0, 16), pl.ds(98304, _TAIL1)],
                            slab0)
            transpose_groups(slab0, outf0, _TAIL1 // 16)
            pltpu.sync_copy(outf0,
                            out_hbm.at[pl.ds(base + 98304 * EMB_DIM,
                                             _TAIL1 * EMB_DIM)])
            pltpu.sync_copy(t2_hbm.at[pl.ds(r0, 16), pl.ds(99840, _TAIL2)],
                            slab1.at[:, pl.ds(0, _TAIL2)])
            transpose_groups(slab1, outf1, _TAIL2 // 16)
            pltpu.sync_copy(outf1.at[pl.ds(0, _TAIL2 * EMB_DIM)],
                            out_hbm.at[pl.ds(base + 99840 * EMB_DIM,
                                             _TAIL2 * EMB_DIM)])
            n = _NREM * EMB_DIM
            pltpu.sync_copy(tail_hbm.at[pl.ds(wid * n, n)],
                            outf0.at[pl.ds(0, n)])
            pltpu.sync_copy(outf0.at[pl.ds(0, n)],
                            out_hbm.at[pl.ds(base + _REM0 * EMB_DIM, n)])

    return _t


@functools.cache
def _make_sc_gather():
    @functools.partial(
        pl.kernel,
        out_type=jax.ShapeDtypeStruct((BATCH * N_FIELDS, EMB_DIM), jnp.float32),
        mesh=plsc.VectorSubcoreMesh(core_axis_name="c", subcore_axis_name="s"),
        scratch_types=[
            pltpu.VMEM((_CHUNK_R,), jnp.int32),
            pltpu.VMEM((_CHUNK_R, EMB_DIM), jnp.float32),
            pltpu.SemaphoreType.DMA,
        ],
        compiler_params=pltpu.CompilerParams(use_tc_tiling_on_sc=False),
    )
    def _sc_gather(table_hbm, idx_hbm, out_hbm, idx_v, rows_v, sem):
        wid = lax.axis_index("s") * 2 + lax.axis_index("c")
        base = wid * (_B_PER_W * N_FIELDS)
        for g in range(_N_CHUNKS):
            off = base + g * _CHUNK_R
            pltpu.sync_copy(idx_hbm.at[pl.ds(off, _CHUNK_R)], idx_v)
            pltpu.async_copy(table_hbm.at[idx_v], rows_v, sem).wait()
            pltpu.sync_copy(rows_v, out_hbm.at[pl.ds(off, _CHUNK_R)])

    return _sc_gather


def _mlp_body(xe_ref, xc_ref, w0e_ref, w0c_ref, b0_ref, w1_ref, b1_ref,
              w2_ref, b2_ref, wh_ref, bh_ref, out_ref):
    h = jnp.dot(xe_ref[...], w0e_ref[...], preferred_element_type=jnp.float32)
    h += jnp.dot(xc_ref[...], w0c_ref[...], preferred_element_type=jnp.float32)
    h = jnp.maximum(h + b0_ref[...], 0.0)
    h = jnp.maximum(
        jnp.dot(h, w1_ref[...], preferred_element_type=jnp.float32) + b1_ref[...], 0.0)
    h = jnp.maximum(
        jnp.dot(h, w2_ref[...], preferred_element_type=jnp.float32) + b2_ref[...], 0.0)
    out_ref[...] = jnp.dot(h, wh_ref[...], preferred_element_type=jnp.float32) + bh_ref[...]


_BT = 1024


def _mlp(xe, xc, w0e, w0c, b0, w1, b1, w2, b2, wh, bh):
    n_blocks = BATCH // _BT
    full = lambda shape: pl.BlockSpec(shape, lambda i: (0, 0))
    return pl.pallas_call(
        _mlp_body,
        grid=(n_blocks,),
        in_specs=[
            pl.BlockSpec((_BT, EMB_FEATS), lambda i: (i, 0)),
            pl.BlockSpec((_BT, N_CONT), lambda i: (i, 0)),
            full((EMB_FEATS, 512)),
            full((N_CONT, 512)),
            full((1, 512)),
            full((512, 256)),
            full((1, 256)),
            full((256, 128)),
            full((1, 128)),
            full((128, 1)),
            full((1, 1)),
        ],
        out_specs=pl.BlockSpec((_BT, 1), lambda i: (i, 0)),
        out_shape=jax.ShapeDtypeStruct((BATCH, 1), jnp.float32),
    )(xe, xc, w0e, w0c, b0, w1, b1, w2, b2, wh, bh)


def kernel(x_cont, x_cat, emb_tables, W0, b0, W1, b1, W2, b2, Wh, bh):
    t2 = jnp.transpose(emb_tables, (0, 2, 1)).reshape(N_FIELDS * EMB_DIM, VOCAB)
    tail = emb_tables[:, _REM0:, :].reshape(-1)
    flat = _make_sc_transpose()(t2, tail)
    table = flat.reshape(N_FIELDS * VOCAB, EMB_DIM)
    offs = jnp.arange(N_FIELDS, dtype=jnp.int32) * VOCAB
    idx = (x_cat.astype(jnp.int32) + offs[None, :]).reshape(-1)
    rows = _make_sc_gather()(table, idx)
    xe = rows.reshape(BATCH, EMB_FEATS)
    return _mlp(xe, x_cont, W0[N_CONT:], W0[:N_CONT],
                b0.reshape(1, -1), W1, b1.reshape(1, -1),
                W2, b2.reshape(1, -1), Wh, bh.reshape(1, 1))


# speedup vs baseline: 3.0713x; 1.0089x over previous
import functools

import jax
import jax.numpy as jnp
from jax import lax
from jax.experimental import pallas as pl
from jax.experimental.pallas import tpu as pltpu
from jax.experimental.pallas import tpu_sc as plsc

N_FIELDS = 26
VOCAB = 100000
EMB_DIM = 16
N_CONT = 13
BATCH = 16384
EMB_FEATS = N_FIELDS * EMB_DIM

_NW = 32

_VS = 1536
_NFULL = 64
_TAIL1 = 1536
_TAIL2 = 128
_REM0 = 99968
_NREM = VOCAB - _REM0

_B_PER_W = BATCH // _NW
_CHUNK_B = 128
_CHUNK_R = _CHUNK_B * N_FIELDS
_N_CHUNKS = _B_PER_W // _CHUNK_B


@functools.cache
def _make_sc_transpose():
    @functools.partial(
        pl.kernel,
        out_type=jax.ShapeDtypeStruct((N_FIELDS * VOCAB * EMB_DIM,), jnp.float32),
        mesh=plsc.VectorSubcoreMesh(core_axis_name="c", subcore_axis_name="s"),
        scratch_types=[
            pltpu.VMEM((EMB_DIM, _VS), jnp.float32),
            pltpu.VMEM((EMB_DIM, _VS), jnp.float32),
            pltpu.VMEM((_VS * EMB_DIM,), jnp.float32),
            pltpu.VMEM((_VS * EMB_DIM,), jnp.float32),
            pltpu.SemaphoreType.DMA,
            pltpu.SemaphoreType.DMA,
            pltpu.SemaphoreType.DMA,
            pltpu.SemaphoreType.DMA,
        ],
        compiler_params=pltpu.CompilerParams(use_tc_tiling_on_sc=True,
                                             needs_layout_passes=False),
    )
    def _t(t2_hbm, tail_hbm, out_hbm, slab0, slab1, outf0, outf1,
           si0, si1, so0, so1):
        wid = lax.axis_index("s") * 2 + lax.axis_index("c")
        lane = lax.iota(jnp.int32, 16)
        lane16 = lane * 16
        v0 = wid * _VS
        v1 = (wid + 32) * _VS
        FB = VOCAB * EMB_DIM

        def transpose_groups(slab, outf, n_groups):
            def body(g, _):
                for e in range(EMB_DIM):
                    rvec = slab[e, pl.ds(g * 16, 16)]
                    plsc.store_scatter(outf, [lane16 + (g * 256 + e)], rvec)
                return 0
            lax.fori_loop(0, n_groups, body, 0, unroll=4)

        def start_in(kk, vv, slab, sem):
            pltpu.async_copy(
                t2_hbm.at[pl.ds(kk * 16, 16), pl.ds(vv, _VS)], slab, sem)

        def wait_in(slab, sem):
            pltpu.make_async_copy(
                t2_hbm.at[pl.ds(0, 16), pl.ds(0, _VS)], slab, sem).wait()

        def wait_out(outf, sem):
            pltpu.make_async_copy(
                outf, out_hbm.at[pl.ds(0, _VS * EMB_DIM)], sem).wait()

        start_in(0, v0, slab0, si0)

        def body(jj, _):
            start_in(jj, v1, slab1, si1)
            wait_in(slab0, si0)
            @pl.when(jj > 0)
            def _():
                wait_out(outf0, so0)
            transpose_groups(slab0, outf0, _VS // 16)
            pltpu.async_copy(
                outf0, out_hbm.at[pl.ds(jj * FB + v0 * EMB_DIM,
                                        _VS * EMB_DIM)], so0)
            @pl.when(jj < N_FIELDS - 1)
            def _():
                start_in(jj + 1, v0, slab0, si0)
            wait_in(slab1, si1)
            @pl.when(jj > 0)
            def _():
                wait_out(outf1, so1)
            transpose_groups(slab1, outf1, _VS // 16)
            pltpu.async_copy(
                outf1, out_hbm.at[pl.ds(jj * FB + v1 * EMB_DIM,
                                        _VS * EMB_DIM)], so1)
            return 0

        lax.fori_loop(0, N_FIELDS, body, 0)
        wait_out(outf0, so0)
        wait_out(outf1, so1)

        @pl.when(wid < N_FIELDS)
        def _():
            base = wid * FB
            r0 = wid * 16
            pltpu.sync_copy(t2_hbm.at[pl.ds(r0, 16), pl.ds(98304, _TAIL1)],
                            slab0)
            transpose_groups(slab0, outf0, _TAIL1 // 16)
            pltpu.sync_copy(outf0,
                            out_hbm.at[pl.ds(base + 98304 * EMB_DIM,
                                             _TAIL1 * EMB_DIM)])
            pltpu.sync_copy(t2_hbm.at[pl.ds(r0, 16), pl.ds(99840, _TAIL2)],
                            slab1.at[:, pl.ds(0, _TAIL2)])
            transpose_groups(slab1, outf1, _TAIL2 // 16)
            pltpu.sync_copy(outf1.at[pl.ds(0, _TAIL2 * EMB_DIM)],
                            out_hbm.at[pl.ds(base + 99840 * EMB_DIM,
                                             _TAIL2 * EMB_DIM)])
            n = _NREM * EMB_DIM
            pltpu.sync_copy(tail_hbm.at[pl.ds(wid * n, n)],
                            outf0.at[pl.ds(0, n)])
            pltpu.sync_copy(outf0.at[pl.ds(0, n)],
                            out_hbm.at[pl.ds(base + _REM0 * EMB_DIM, n)])

    return _t


@functools.cache
def _make_sc_gather(nbatch):
    n_chunks = nbatch // _NW // _CHUNK_B
    @functools.partial(
        pl.kernel,
        out_type=jax.ShapeDtypeStruct((nbatch * N_FIELDS, EMB_DIM), jnp.float32),
        mesh=plsc.VectorSubcoreMesh(core_axis_name="c", subcore_axis_name="s"),
        scratch_types=[
            pltpu.VMEM((_CHUNK_R,), jnp.int32),
            pltpu.VMEM((_CHUNK_R, EMB_DIM), jnp.float32),
            pltpu.SemaphoreType.DMA,
        ],
        compiler_params=pltpu.CompilerParams(use_tc_tiling_on_sc=False),
    )
    def _sc_gather(table_hbm, idx_hbm, out_hbm, idx_v, rows_v, sem):
        wid = lax.axis_index("s") * 2 + lax.axis_index("c")
        base = wid * (n_chunks * _CHUNK_R)
        for g in range(n_chunks):
            off = base + g * _CHUNK_R
            pltpu.sync_copy(idx_hbm.at[pl.ds(off, _CHUNK_R)], idx_v)
            pltpu.async_copy(table_hbm.at[idx_v], rows_v, sem).wait()
            pltpu.sync_copy(rows_v, out_hbm.at[pl.ds(off, _CHUNK_R)])

    return _sc_gather


def _mlp_body(xe_ref, xc_ref, w0e_ref, w0c_ref, b0_ref, w1_ref, b1_ref,
              w2_ref, b2_ref, wh_ref, bh_ref, out_ref):
    h = jnp.dot(xe_ref[...], w0e_ref[...], preferred_element_type=jnp.float32)
    h += jnp.dot(xc_ref[...], w0c_ref[...], preferred_element_type=jnp.float32)
    h = jnp.maximum(h + b0_ref[...], 0.0)
    h = jnp.maximum(
        jnp.dot(h, w1_ref[...], preferred_element_type=jnp.float32) + b1_ref[...], 0.0)
    h = jnp.maximum(
        jnp.dot(h, w2_ref[...], preferred_element_type=jnp.float32) + b2_ref[...], 0.0)
    out_ref[...] = jnp.dot(h, wh_ref[...], preferred_element_type=jnp.float32) + bh_ref[...]


_BT = 1024


def _mlp(xe, xc, w0e, w0c, b0, w1, b1, w2, b2, wh, bh):
    nbatch = xe.shape[0]
    n_blocks = nbatch // _BT
    full = lambda shape: pl.BlockSpec(shape, lambda i: (0, 0))
    return pl.pallas_call(
        _mlp_body,
        grid=(n_blocks,),
        in_specs=[
            pl.BlockSpec((_BT, EMB_FEATS), lambda i: (i, 0)),
            pl.BlockSpec((_BT, N_CONT), lambda i: (i, 0)),
            full((EMB_FEATS, 512)),
            full((N_CONT, 512)),
            full((1, 512)),
            full((512, 256)),
            full((1, 256)),
            full((256, 128)),
            full((1, 128)),
            full((128, 1)),
            full((1, 1)),
        ],
        out_specs=pl.BlockSpec((_BT, 1), lambda i: (i, 0)),
        out_shape=jax.ShapeDtypeStruct((nbatch, 1), jnp.float32),
    )(xe, xc, w0e, w0c, b0, w1, b1, w2, b2, wh, bh)


def kernel(x_cont, x_cat, emb_tables, W0, b0, W1, b1, W2, b2, Wh, bh):
    t2 = jnp.transpose(emb_tables, (0, 2, 1)).reshape(N_FIELDS * EMB_DIM, VOCAB)
    tail = emb_tables[:, _REM0:, :].reshape(-1)
    flat = _make_sc_transpose()(t2, tail)
    table = flat.reshape(N_FIELDS * VOCAB, EMB_DIM)
    offs = jnp.arange(N_FIELDS, dtype=jnp.int32) * VOCAB
    idx = (x_cat.astype(jnp.int32) + offs[None, :]).reshape(-1)
    hb = BATCH // 2
    w0e, w0c = W0[N_CONT:], W0[:N_CONT]
    b0r, b1r, b2r, bhr = (b0.reshape(1, -1), b1.reshape(1, -1),
                          b2.reshape(1, -1), bh.reshape(1, 1))
    outs = []
    for h in range(2):
        rows_h = _make_sc_gather(hb)(table, lax.slice_in_dim(
            idx, h * hb * N_FIELDS, (h + 1) * hb * N_FIELDS))
        xe_h = rows_h.reshape(hb, EMB_FEATS)
        outs.append(_mlp(xe_h, lax.slice_in_dim(x_cont, h * hb, (h + 1) * hb),
                         w0e, w0c, b0r, W1, b1r, W2, b2r, Wh, bhr))
    return jnp.concatenate(outs, axis=0)
